# baseline (speedup 1.0000x reference)
import jax
import jax.numpy as jnp
from jax import lax
from jax.experimental import pallas as pl
from jax.experimental.pallas import tpu as pltpu
from jax.experimental.pallas import tpu_sc as plsc

S, B, K, D = 8, 1024, 64, 32
KB = 32
NKB = K // KB
BT = 640
BS = B - BT
NC, NS, L = 2, 16, 16
CW = 128
NW_SC = S * (BS // CW)
KG = 4
NG = K // KG


def _select_body(z_ref, mu_ref, sg_ref, p_ref, out_ref, acc_sg, zb):
  kb = pl.program_id(1)

  @pl.when(kb == 0)
  def _bcast():
    zb[...] = jnp.broadcast_to(z_ref[0, 0, :][None, :], (D, BT))

  zt = zb[...]
  acc_mu = out_ref[0]
  acc_s = acc_sg[...]
  for kk in range(KB):
    m = zt == (kb * KB + kk)
    acc_mu = jnp.where(m, mu_ref[0, kk], acc_mu)
    acc_s = jnp.where(m, sg_ref[0, kk], acc_s)

  @pl.when(kb < NKB - 1)
  def _store():
    out_ref[0] = acc_mu
    acc_sg[...] = acc_s

  @pl.when(kb == NKB - 1)
  def _fin():
    out_ref[0] = acc_mu + acc_s * p_ref[0]


@jax.jit
def _sample_point_tc(mus_t, sig_t, p_t, z3):
  return pl.pallas_call(
      _select_body,
      grid=(S, NKB),
      in_specs=[
          pl.BlockSpec((1, 1, BT), lambda s, kb: (s, 0, 0)),
          pl.BlockSpec((1, KB, D, BT), lambda s, kb: (s, kb, 0, 0)),
          pl.BlockSpec((1, KB, D, BT), lambda s, kb: (s, kb, 0, 0)),
          pl.BlockSpec((1, D, BT), lambda s, kb: (s, 0, 0)),
      ],
      out_specs=pl.BlockSpec((1, D, BT), lambda s, kb: (s, 0, 0)),
      out_shape=jax.ShapeDtypeStruct((S, D, BT), jnp.float32),
      scratch_shapes=[pltpu.VMEM((D, BT), jnp.float32),
                      pltpu.VMEM((D, BT), jnp.int32)],
      compiler_params=pltpu.CompilerParams(
          dimension_semantics=("parallel", "arbitrary")),
  )(z3, mus_t, sig_t, p_t)


def _sc_body(mu_hbm, sg_hbm, p_hbm, z_hbm, out_hbm,
             z_buf, mu_buf, sg_buf, p_buf, acc_mu, acc_sg, sem_mu, sem_sg):
  wid = lax.axis_index("s") * NC + lax.axis_index("c")

  @pl.when(wid < NW_SC)
  def _work():
    s = wid // (BS // CW)
    c = wid % (BS // CW)
    b0 = BT + c * CW
    r_base = s * (K * D)

    pltpu.sync_copy(z_hbm.at[s, pl.ds(b0, CW)], z_buf)
    pltpu.sync_copy(p_hbm.at[pl.ds(s * D, D), pl.ds(b0, CW)], p_buf)

    def fire(g, par):
      rows = pl.ds(r_base + g * (KG * D), KG * D)
      cols = pl.ds(b0, CW)
      pltpu.async_copy(mu_hbm.at[rows, cols], mu_buf.at[par], sem_mu)
      pltpu.async_copy(sg_hbm.at[rows, cols], sg_buf.at[par], sem_sg)

    fire(0, 0)

    def group(g, _):
      par = lax.rem(g, 2)
      pltpu.make_async_copy(
          mu_hbm.at[pl.ds(0, KG * D), pl.ds(0, CW)], mu_buf.at[par], sem_mu
      ).wait()
      pltpu.make_async_copy(
          sg_hbm.at[pl.ds(0, KG * D), pl.ds(0, CW)], sg_buf.at[par], sem_sg
      ).wait()

      @pl.when(g < NG - 1)
      def _prefetch():
        fire(g + 1, 1 - par)

      def drow(d, _):
        for v in range(CW // L):
          sl = pl.ds(v * L, L)
          zv = z_buf[sl]
          amu = acc_mu[d, sl]
          asg = acc_sg[d, sl]
          for kk in range(KG):
            m = zv == (g * KG + kk)
            amu = jnp.where(m, mu_buf[par, kk * D + d, sl], amu)
            asg = jnp.where(m, sg_buf[par, kk * D + d, sl], asg)
          acc_mu[d, sl] = amu
          acc_sg[d, sl] = asg
        return 0

      lax.fori_loop(0, D, drow, 0)
      return 0

    lax.fori_loop(0, NG, group, 0)

    def frow(d, _):
      for v in range(CW // L):
        sl = pl.ds(v * L, L)
        acc_mu[d, sl] = acc_mu[d, sl] + acc_sg[d, sl] * p_buf[d, sl]
      return 0

    lax.fori_loop(0, D, frow, 0)
    pltpu.sync_copy(acc_mu,
                    out_hbm.at[pl.ds(s * D, D), pl.ds(c * CW, CW)])


@jax.jit
def _sample_point_sc(mus2d, sig2d, p2d, z):
  mesh = plsc.VectorSubcoreMesh(core_axis_name="c", subcore_axis_name="s",
                                num_cores=NC, num_subcores=NS)
  run = pl.kernel(
      _sc_body,
      out_type=jax.ShapeDtypeStruct((S * D, BS), jnp.float32),
      mesh=mesh,
      scratch_types=[
          pltpu.VMEM((CW,), jnp.int32),
          pltpu.VMEM((2, KG * D, CW), jnp.float32),
          pltpu.VMEM((2, KG * D, CW), jnp.float32),
          pltpu.VMEM((D, CW), jnp.float32),
          pltpu.VMEM((D, CW), jnp.float32),
          pltpu.VMEM((D, CW), jnp.float32),
          pltpu.SemaphoreType.DMA,
          pltpu.SemaphoreType.DMA,
      ],
  )
  return run(mus2d, sig2d, p2d, z)


def kernel(p, mus, sigmas, z):
  mus_t = mus.transpose(0, 2, 3, 1)
  sig_t = sigmas.transpose(0, 2, 3, 1)
  p_t = p.transpose(0, 2, 1)
  zi = z.astype(jnp.int32)
  z3 = zi.reshape(S, 1, B)
  sc_out = _sample_point_sc(mus_t.reshape(S * K * D, B),
                            sig_t.reshape(S * K * D, B),
                            p_t.reshape(S * D, B), zi)
  tc_out = _sample_point_tc(mus_t, sig_t, p_t, z3)
  out_t = jnp.concatenate([tc_out, sc_out.reshape(S, D, BS)], axis=2)
  return out_t.transpose(0, 2, 1)


# speedup vs baseline: 1.0886x; 1.0886x over previous
import jax
import jax.numpy as jnp
from jax import lax
from jax.experimental import pallas as pl
from jax.experimental.pallas import tpu as pltpu
from jax.experimental.pallas import tpu_sc as plsc

S, B, K, D = 8, 1024, 64, 32
KB = 32
NKB = K // KB
BT = 768
BS = B - BT
NC, NS, L = 2, 16, 16
CW = 128
KG = 4
KH = K // 2
NGH = KH // KG


def _select_body(z_ref, mu_ref, sg_ref, p_ref, out_ref, acc_sg, zb):
  kb = pl.program_id(1)

  @pl.when(kb == 0)
  def _bcast():
    zb[...] = jnp.broadcast_to(z_ref[0, 0, :][None, :], (D, BT))

  zt = zb[...]
  acc_mu = out_ref[0]
  acc_s = acc_sg[...]
  for kk in range(KB):
    m = zt == (kb * KB + kk)
    acc_mu = jnp.where(m, mu_ref[0, kk], acc_mu)
    acc_s = jnp.where(m, sg_ref[0, kk], acc_s)

  @pl.when(kb < NKB - 1)
  def _store():
    out_ref[0] = acc_mu
    acc_sg[...] = acc_s

  @pl.when(kb == NKB - 1)
  def _fin():
    out_ref[0] = acc_mu + acc_s * p_ref[0]


@jax.jit
def _sample_point_tc(mus_t, sig_t, p_t, z3):
  return pl.pallas_call(
      _select_body,
      grid=(S, NKB),
      in_specs=[
          pl.BlockSpec((1, 1, BT), lambda s, kb: (s, 0, 0)),
          pl.BlockSpec((1, KB, D, BT), lambda s, kb: (s, kb, 0, 0)),
          pl.BlockSpec((1, KB, D, BT), lambda s, kb: (s, kb, 0, 0)),
          pl.BlockSpec((1, D, BT), lambda s, kb: (s, 0, 0)),
      ],
      out_specs=pl.BlockSpec((1, D, BT), lambda s, kb: (s, 0, 0)),
      out_shape=jax.ShapeDtypeStruct((S, D, BT), jnp.float32),
      scratch_shapes=[pltpu.VMEM((D, BT), jnp.float32),
                      pltpu.VMEM((D, BT), jnp.int32)],
      compiler_params=pltpu.CompilerParams(
          dimension_semantics=("parallel", "arbitrary")),
  )(z3, mus_t, sig_t, p_t)


def _sc_body(mu_hbm, sg_hbm, p_hbm, z_hbm, out_hbm,
             z_buf, mu_buf, sg_buf, p_buf, acc_mu, acc_sg, sem_mu, sem_sg):
  wid = lax.axis_index("s") * NC + lax.axis_index("c")
  s = wid // 4
  r = wid % 4
  c = r // 2
  h = r % 2
  b0 = BT + c * CW
  r_base = s * (K * D) + h * (KH * D)

  pltpu.sync_copy(z_hbm.at[s, pl.ds(b0, CW)], z_buf)
  pltpu.sync_copy(p_hbm.at[pl.ds(s * D, D), pl.ds(b0, CW)], p_buf)

  def zrow(d, _):
    for v in range(CW // L):
      sl = pl.ds(v * L, L)
      acc_mu[d, sl] = jnp.zeros((L,), jnp.float32)
      acc_sg[d, sl] = jnp.zeros((L,), jnp.float32)
    return 0

  lax.fori_loop(0, D, zrow, 0)

  def fire(g, par):
    rows = pl.ds(r_base + g * (KG * D), KG * D)
    cols = pl.ds(b0, CW)
    pltpu.async_copy(mu_hbm.at[rows, cols], mu_buf.at[par], sem_mu)
    pltpu.async_copy(sg_hbm.at[rows, cols], sg_buf.at[par], sem_sg)

  fire(0, 0)

  def group(g, _):
    par = lax.rem(g, 2)
    pltpu.make_async_copy(
        mu_hbm.at[pl.ds(0, KG * D), pl.ds(0, CW)], mu_buf.at[par], sem_mu
    ).wait()
    pltpu.make_async_copy(
        sg_hbm.at[pl.ds(0, KG * D), pl.ds(0, CW)], sg_buf.at[par], sem_sg
    ).wait()

    @pl.when(g < NGH - 1)
    def _prefetch():
      fire(g + 1, 1 - par)

    k0 = h * KH + g * KG

    def drow(d, _):
      for v in range(CW // L):
        sl = pl.ds(v * L, L)
        zv = z_buf[sl]
        amu = acc_mu[d, sl]
        asg = acc_sg[d, sl]
        for kk in range(KG):
          m = zv == (k0 + kk)
          amu = jnp.where(m, mu_buf[par, kk * D + d, sl], amu)
          asg = jnp.where(m, sg_buf[par, kk * D + d, sl], asg)
        acc_mu[d, sl] = amu
        acc_sg[d, sl] = asg
      return 0

    lax.fori_loop(0, D, drow, 0)
    return 0

  lax.fori_loop(0, NGH, group, 0)

  def frow(d, _):
    for v in range(CW // L):
      sl = pl.ds(v * L, L)
      acc_mu[d, sl] = acc_mu[d, sl] + acc_sg[d, sl] * p_buf[d, sl]
    return 0

  lax.fori_loop(0, D, frow, 0)
  pltpu.sync_copy(acc_mu,
                  out_hbm.at[h, pl.ds(s * D, D), pl.ds(c * CW, CW)])


@jax.jit
def _sample_point_sc(mus2d, sig2d, p2d, z):
  mesh = plsc.VectorSubcoreMesh(core_axis_name="c", subcore_axis_name="s",
                                num_cores=NC, num_subcores=NS)
  run = pl.kernel(
      _sc_body,
      out_type=jax.ShapeDtypeStruct((2, S * D, BS), jnp.float32),
      mesh=mesh,
      scratch_types=[
          pltpu.VMEM((CW,), jnp.int32),
          pltpu.VMEM((2, KG * D, CW), jnp.float32),
          pltpu.VMEM((2, KG * D, CW), jnp.float32),
          pltpu.VMEM((D, CW), jnp.float32),
          pltpu.VMEM((D, CW), jnp.float32),
          pltpu.VMEM((D, CW), jnp.float32),
          pltpu.SemaphoreType.DMA,
          pltpu.SemaphoreType.DMA,
      ],
  )
  return run(mus2d, sig2d, p2d, z)


def kernel(p, mus, sigmas, z):
  mus_t = mus.transpose(0, 2, 3, 1)
  sig_t = sigmas.transpose(0, 2, 3, 1)
  p_t = p.transpose(0, 2, 1)
  zi = z.astype(jnp.int32)
  z3 = zi.reshape(S, 1, B)
  sc_parts = _sample_point_sc(mus_t.reshape(S * K * D, B),
                              sig_t.reshape(S * K * D, B),
                              p_t.reshape(S * D, B), zi)
  tc_out = _sample_point_tc(mus_t, sig_t, p_t, z3)
  sc_out = (sc_parts[0] + sc_parts[1]).reshape(S, D, BS)
  out_t = jnp.concatenate([tc_out, sc_out], axis=2)
  return out_t.transpose(0, 2, 1)


# speedup vs baseline: 1.0941x; 1.0050x over previous
import jax
import jax.numpy as jnp
from jax import lax
from jax.experimental import pallas as pl
from jax.experimental.pallas import tpu as pltpu
from jax.experimental.pallas import tpu_sc as plsc

S, B, K, D = 8, 1024, 64, 32
KB = 32
NKB = K // KB
BT = 896
BS = B - BT
NC, NS, L = 2, 16, 16
CW = 128
KG = 4
KH = K // 4
NGH = KH // KG


def _select_body(z_ref, mu_ref, sg_ref, p_ref, out_ref, acc_sg, zb):
  kb = pl.program_id(1)

  @pl.when(kb == 0)
  def _bcast():
    zb[...] = jnp.broadcast_to(z_ref[0, 0, :][None, :], (D, BT))

  zt = zb[...]
  acc_mu = out_ref[0]
  acc_s = acc_sg[...]
  for kk in range(KB):
    m = zt == (kb * KB + kk)
    acc_mu = jnp.where(m, mu_ref[0, kk], acc_mu)
    acc_s = jnp.where(m, sg_ref[0, kk], acc_s)

  @pl.when(kb < NKB - 1)
  def _store():
    out_ref[0] = acc_mu
    acc_sg[...] = acc_s

  @pl.when(kb == NKB - 1)
  def _fin():
    out_ref[0] = acc_mu + acc_s * p_ref[0]


@jax.jit
def _sample_point_tc(mus_t, sig_t, p_t, z3):
  return pl.pallas_call(
      _select_body,
      grid=(S, NKB),
      in_specs=[
          pl.BlockSpec((1, 1, BT), lambda s, kb: (s, 0, 0)),
          pl.BlockSpec((1, KB, D, BT), lambda s, kb: (s, kb, 0, 0)),
          pl.BlockSpec((1, KB, D, BT), lambda s, kb: (s, kb, 0, 0)),
          pl.BlockSpec((1, D, BT), lambda s, kb: (s, 0, 0)),
      ],
      out_specs=pl.BlockSpec((1, D, BT), lambda s, kb: (s, 0, 0)),
      out_shape=jax.ShapeDtypeStruct((S, D, BT), jnp.float32),
      scratch_shapes=[pltpu.VMEM((D, BT), jnp.float32),
                      pltpu.VMEM((D, BT), jnp.int32)],
      compiler_params=pltpu.CompilerParams(
          dimension_semantics=("parallel", "arbitrary")),
  )(z3, mus_t, sig_t, p_t)


def _sc_body(mu_hbm, sg_hbm, p_hbm, z_hbm, out_hbm,
             z_buf, mu_buf, sg_buf, p_buf, acc_mu, acc_sg, sem_mu, sem_sg):
  wid = lax.axis_index("s") * NC + lax.axis_index("c")
  s = wid // 4
  h = wid % 4
  b0 = BT
  r_base = s * (K * D) + h * (KH * D)

  pltpu.sync_copy(z_hbm.at[s, pl.ds(b0, CW)], z_buf)
  pltpu.sync_copy(p_hbm.at[pl.ds(s * D, D), pl.ds(b0, CW)], p_buf)

  def zrow(d, _):
    for v in range(CW // L):
      sl = pl.ds(v * L, L)
      acc_mu[d, sl] = jnp.zeros((L,), jnp.float32)
      acc_sg[d, sl] = jnp.zeros((L,), jnp.float32)
    return 0

  lax.fori_loop(0, D, zrow, 0)

  def fire(g, par):
    rows = pl.ds(r_base + g * (KG * D), KG * D)
    cols = pl.ds(b0, CW)
    pltpu.async_copy(mu_hbm.at[rows, cols], mu_buf.at[par], sem_mu)
    pltpu.async_copy(sg_hbm.at[rows, cols], sg_buf.at[par], sem_sg)

  fire(0, 0)

  def group(g, _):
    par = lax.rem(g, 2)
    pltpu.make_async_copy(
        mu_hbm.at[pl.ds(0, KG * D), pl.ds(0, CW)], mu_buf.at[par], sem_mu
    ).wait()
    pltpu.make_async_copy(
        sg_hbm.at[pl.ds(0, KG * D), pl.ds(0, CW)], sg_buf.at[par], sem_sg
    ).wait()

    @pl.when(g < NGH - 1)
    def _prefetch():
      fire(g + 1, 1 - par)

    k0 = h * KH + g * KG

    def drow(d, _):
      for v in range(CW // L):
        sl = pl.ds(v * L, L)
        zv = z_buf[sl]
        amu = acc_mu[d, sl]
        asg = acc_sg[d, sl]
        for kk in range(KG):
          m = zv == (k0 + kk)
          amu = jnp.where(m, mu_buf[par, kk * D + d, sl], amu)
          asg = jnp.where(m, sg_buf[par, kk * D + d, sl], asg)
        acc_mu[d, sl] = amu
        acc_sg[d, sl] = asg
      return 0

    lax.fori_loop(0, D, drow, 0)
    return 0

  lax.fori_loop(0, NGH, group, 0)

  def frow(d, _):
    for v in range(CW // L):
      sl = pl.ds(v * L, L)
      acc_mu[d, sl] = acc_mu[d, sl] + acc_sg[d, sl] * p_buf[d, sl]
    return 0

  lax.fori_loop(0, D, frow, 0)
  pltpu.sync_copy(acc_mu,
                  out_hbm.at[h, pl.ds(s * D, D), pl.ds(0, CW)])


@jax.jit
def _sample_point_sc(mus2d, sig2d, p2d, z):
  mesh = plsc.VectorSubcoreMesh(core_axis_name="c", subcore_axis_name="s",
                                num_cores=NC, num_subcores=NS)
  run = pl.kernel(
      _sc_body,
      out_type=jax.ShapeDtypeStruct((4, S * D, BS), jnp.float32),
      mesh=mesh,
      scratch_types=[
          pltpu.VMEM((CW,), jnp.int32),
          pltpu.VMEM((2, KG * D, CW), jnp.float32),
          pltpu.VMEM((2, KG * D, CW), jnp.float32),
          pltpu.VMEM((D, CW), jnp.float32),
          pltpu.VMEM((D, CW), jnp.float32),
          pltpu.VMEM((D, CW), jnp.float32),
          pltpu.SemaphoreType.DMA,
          pltpu.SemaphoreType.DMA,
      ],
  )
  return run(mus2d, sig2d, p2d, z)


def kernel(p, mus, sigmas, z):
  mus_t = mus.transpose(0, 2, 3, 1)
  sig_t = sigmas.transpose(0, 2, 3, 1)
  p_t = p.transpose(0, 2, 1)
  zi = z.astype(jnp.int32)
  z3 = zi.reshape(S, 1, B)
  sc_parts = _sample_point_sc(mus_t.reshape(S * K * D, B),
                              sig_t.reshape(S * K * D, B),
                              p_t.reshape(S * D, B), zi)
  tc_out = _sample_point_tc(mus_t, sig_t, p_t, z3)
  sc_out = (sc_parts[0] + sc_parts[1]
            + sc_parts[2] + sc_parts[3]).reshape(S, D, BS)
  out_t = jnp.concatenate([tc_out, sc_out], axis=2)
  return out_t.transpose(0, 2, 1)


# speedup vs baseline: 1.5770x; 1.4414x over previous
import jax
import jax.numpy as jnp
from jax.experimental import pallas as pl
from jax.experimental.pallas import tpu as pltpu

S, B, K, D = 8, 1024, 64, 32
KB = 32
NKB = K // KB


def _select_body(z_ref, mu_ref, sg_ref, p_ref, out_ref, acc_sg, zb):
  kb = pl.program_id(1)

  @pl.when(kb == 0)
  def _bcast():
    zb[...] = jnp.broadcast_to(z_ref[0, 0, :][None, :], (D, B))

  zt = zb[...]
  acc_mu = out_ref[0]
  acc_s = acc_sg[...]
  for kk in range(KB):
    m = zt == (kb * KB + kk)
    acc_mu = jnp.where(m, mu_ref[0, kk], acc_mu)
    acc_s = jnp.where(m, sg_ref[0, kk], acc_s)

  @pl.when(kb < NKB - 1)
  def _store():
    out_ref[0] = acc_mu
    acc_sg[...] = acc_s

  @pl.when(kb == NKB - 1)
  def _fin():
    out_ref[0] = acc_mu + acc_s * p_ref[0]


@jax.jit
def _sample_point_tc(mus_t, sig_t, p_t, z3):
  return pl.pallas_call(
      _select_body,
      grid=(S, NKB),
      in_specs=[
          pl.BlockSpec((1, 1, B), lambda s, kb: (s, 0, 0)),
          pl.BlockSpec((1, KB, D, B), lambda s, kb: (s, kb, 0, 0)),
          pl.BlockSpec((1, KB, D, B), lambda s, kb: (s, kb, 0, 0)),
          pl.BlockSpec((1, D, B), lambda s, kb: (s, 0, 0)),
      ],
      out_specs=pl.BlockSpec((1, D, B), lambda s, kb: (s, 0, 0)),
      out_shape=jax.ShapeDtypeStruct((S, D, B), jnp.float32),
      scratch_shapes=[pltpu.VMEM((D, B), jnp.float32),
                      pltpu.VMEM((D, B), jnp.int32)],
      compiler_params=pltpu.CompilerParams(
          dimension_semantics=("parallel", "arbitrary")),
  )(z3, mus_t, sig_t, p_t)


def kernel(p, mus, sigmas, z):
  mus_t = mus.transpose(0, 2, 3, 1)
  sig_t = sigmas.transpose(0, 2, 3, 1)
  p_t = p.transpose(0, 2, 1)
  z3 = z.reshape(S, 1, B).astype(jnp.int32)
  out_t = _sample_point_tc(mus_t, sig_t, p_t, z3)
  return out_t.transpose(0, 2, 1)
